# Initial kernel scaffold; baseline (speedup 1.0000x reference)
import jax
import jax.numpy as jnp
from jax.experimental import pallas as pl


def kernel(r_s, weight_W, bias, edge_index):
    raise NotImplementedError("write your pallas kernel here")


# speedup vs baseline: 1.0512x; 1.0512x over previous
import functools

import jax
import jax.numpy as jnp
from jax import lax
from jax.experimental import pallas as pl
from jax.experimental.pallas import tpu as pltpu


def _edge_kernel(r_ref, wk_ref, row_ref, col_ref, out_ref, v_ref, agg_ref,
                 *, n_steps):
    e = pl.program_id(1)
    n_nodes = r_ref.shape[0]
    e_tile = row_ref.shape[1]

    @pl.when(e == 0)
    def _init():
        v_ref[...] = jnp.dot(r_ref[...], wk_ref[...],
                             preferred_element_type=jnp.float32)
        agg_ref[...] = jnp.zeros_like(agg_ref)

    iota = lax.broadcasted_iota(jnp.int32, (n_nodes, e_tile), 0)
    row_oh = (iota == row_ref[...]).astype(jnp.float32)
    col_oh = (iota == col_ref[...]).astype(jnp.float32)

    e_vals = lax.dot_general(col_oh, v_ref[...],
                             dimension_numbers=(((0,), (0,)), ((), ())),
                             preferred_element_type=jnp.float32)
    agg_ref[...] += lax.dot_general(row_oh, e_vals,
                                    dimension_numbers=(((1,), (0,)), ((), ())),
                                    preferred_element_type=jnp.float32)

    @pl.when(e == n_steps - 1)
    def _emit():
        out_ref[...] = agg_ref[...][None]


def _finalize_kernel(r_ref, w_ref, b_ref, agg2_ref, out_ref, *, k, k_pad):
    n_nodes = r_ref.shape[0]
    v = jnp.dot(r_ref[...], w_ref[...],
                preferred_element_type=jnp.float32)
    agg = agg2_ref[0] + agg2_ref[1]
    lane = lax.broadcasted_iota(jnp.int32, (n_nodes, k_pad), 1)
    head = jnp.where(lane < k, agg, v[:, 0:k_pad])
    if k_pad < v.shape[1]:
        combined = jnp.concatenate([head, v[:, k_pad:]], axis=1)
    else:
        combined = head
    combined = combined + b_ref[...]
    i_s = jnp.maximum(combined, jnp.float32(0.0))
    out_ref[...] = jnp.mean(i_s, axis=0, keepdims=True)


@jax.jit
def _forward(r_s, weight_W, bias, edge_index):
    N, F = r_s.shape
    P = weight_W.shape[1]
    E = edge_index.shape[1]
    k = P // 10

    P_pad = ((P + 127) // 128) * 128
    k_pad = min(((max(k, 1) + 127) // 128) * 128, P_pad)

    E_TILE = 512
    n_steps = pl.cdiv(pl.cdiv(E, E_TILE), 2)
    E_pad = 2 * n_steps * E_TILE

    r32 = r_s.astype(jnp.float32)
    w = jnp.zeros((F, P_pad), jnp.float32).at[:, :P].set(
        weight_W.astype(jnp.float32))
    b = jnp.zeros((1, P_pad), jnp.float32).at[:, :P].set(
        bias.astype(jnp.float32)[None, :])
    row = jnp.full((1, E_pad), -1, jnp.int32).at[:, :E].set(
        edge_index[0].astype(jnp.int32)[None, :])
    col = jnp.full((1, E_pad), -1, jnp.int32).at[:, :E].set(
        edge_index[1].astype(jnp.int32)[None, :])

    agg2 = pl.pallas_call(
        functools.partial(_edge_kernel, n_steps=n_steps),
        out_shape=jax.ShapeDtypeStruct((2, N, k_pad), jnp.float32),
        grid_spec=pltpu.PrefetchScalarGridSpec(
            num_scalar_prefetch=0,
            grid=(2, n_steps),
            in_specs=[
                pl.BlockSpec((N, F), lambda c, e: (0, 0)),
                pl.BlockSpec((F, k_pad), lambda c, e: (0, 0)),
                pl.BlockSpec((1, E_TILE),
                             lambda c, e, n=n_steps: (0, c * n + e)),
                pl.BlockSpec((1, E_TILE),
                             lambda c, e, n=n_steps: (0, c * n + e)),
            ],
            out_specs=pl.BlockSpec((1, N, k_pad), lambda c, e: (c, 0, 0)),
            scratch_shapes=[
                pltpu.VMEM((N, k_pad), jnp.float32),
                pltpu.VMEM((N, k_pad), jnp.float32),
            ],
        ),
        compiler_params=pltpu.CompilerParams(
            dimension_semantics=("parallel", "arbitrary"),
        ),
    )(r32, w[:, :k_pad], row, col)

    out = pl.pallas_call(
        functools.partial(_finalize_kernel, k=k, k_pad=k_pad),
        out_shape=jax.ShapeDtypeStruct((1, P_pad), jnp.float32),
        in_specs=[
            pl.BlockSpec((N, F), lambda: (0, 0)),
            pl.BlockSpec((F, P_pad), lambda: (0, 0)),
            pl.BlockSpec((1, P_pad), lambda: (0, 0)),
            pl.BlockSpec((2, N, k_pad), lambda: (0, 0, 0)),
        ],
        out_specs=pl.BlockSpec((1, P_pad), lambda: (0, 0)),
        grid=(),
    )(r32, w, b, agg2)
    return out[0, :P]


def kernel(r_s, weight_W, bias, edge_index):
    return _forward(r_s, weight_W, bias, edge_index)


# speedup vs baseline: 1.9030x; 1.8103x over previous
import functools

import numpy as np

import jax
import jax.numpy as jnp
from jax import lax
from jax.experimental import pallas as pl
from jax.experimental.pallas import tpu as pltpu
from jax.sharding import Mesh, PartitionSpec as PSpec

try:
    from jax.experimental.shard_map import shard_map as _shard_map
except ImportError:
    _shard_map = jax.shard_map


def _edge_kernel(r_ref, wk_ref, row_ref, col_ref, out_ref, v_ref, agg_ref,
                 *, n_steps):
    e = pl.program_id(0)
    n_nodes = r_ref.shape[0]
    e_tile = row_ref.shape[1]

    @pl.when(e == 0)
    def _init():
        v_ref[...] = jnp.dot(r_ref[...], wk_ref[...],
                             preferred_element_type=jnp.float32)
        agg_ref[...] = jnp.zeros_like(agg_ref)

    iota = lax.broadcasted_iota(jnp.int32, (n_nodes, e_tile), 0)
    row_oh = (iota == row_ref[...]).astype(jnp.float32)
    col_oh = (iota == col_ref[...]).astype(jnp.float32)

    e_vals = lax.dot_general(col_oh, v_ref[...],
                             dimension_numbers=(((0,), (0,)), ((), ())),
                             preferred_element_type=jnp.float32)
    agg_ref[...] += lax.dot_general(row_oh, e_vals,
                                    dimension_numbers=(((1,), (0,)), ((), ())),
                                    preferred_element_type=jnp.float32)

    @pl.when(e == n_steps - 1)
    def _emit():
        out_ref[...] = agg_ref[...]


def _finalize_kernel(r_ref, w_ref, b_ref, agg_ref, out_ref, *, k, k_pad):
    n_nodes = r_ref.shape[0]
    n_parts = agg_ref.shape[0]
    v = jnp.dot(r_ref[...], w_ref[...],
                preferred_element_type=jnp.float32)
    agg = agg_ref[0]
    for i in range(1, n_parts):
        agg = agg + agg_ref[i]
    lane = lax.broadcasted_iota(jnp.int32, (n_nodes, k_pad), 1)
    head = jnp.where(lane < k, agg, v[:, 0:k_pad])
    if k_pad < v.shape[1]:
        combined = jnp.concatenate([head, v[:, k_pad:]], axis=1)
    else:
        combined = head
    combined = combined + b_ref[...]
    i_s = jnp.maximum(combined, jnp.float32(0.0))
    out_ref[...] = jnp.mean(i_s, axis=0, keepdims=True)


def _edge_call(r32, wk, row_sh, col_sh, *, N, F, k_pad, E_TILE, n_steps):
    return pl.pallas_call(
        functools.partial(_edge_kernel, n_steps=n_steps),
        out_shape=jax.ShapeDtypeStruct((N, k_pad), jnp.float32),
        grid_spec=pltpu.PrefetchScalarGridSpec(
            num_scalar_prefetch=0,
            grid=(n_steps,),
            in_specs=[
                pl.BlockSpec((N, F), lambda e: (0, 0)),
                pl.BlockSpec((F, k_pad), lambda e: (0, 0)),
                pl.BlockSpec((1, E_TILE), lambda e: (0, e)),
                pl.BlockSpec((1, E_TILE), lambda e: (0, e)),
            ],
            out_specs=pl.BlockSpec((N, k_pad), lambda e: (0, 0)),
            scratch_shapes=[
                pltpu.VMEM((N, k_pad), jnp.float32),
                pltpu.VMEM((N, k_pad), jnp.float32),
            ],
        ),
        compiler_params=pltpu.CompilerParams(
            dimension_semantics=("arbitrary",),
        ),
    )(r32, wk, row_sh, col_sh)


def _finalize_call(r32, w, b, agg_parts, *, N, F, P_pad, k, k_pad):
    n_parts = agg_parts.shape[0]
    return pl.pallas_call(
        functools.partial(_finalize_kernel, k=k, k_pad=k_pad),
        out_shape=jax.ShapeDtypeStruct((1, P_pad), jnp.float32),
        in_specs=[
            pl.BlockSpec((N, F), lambda: (0, 0)),
            pl.BlockSpec((F, P_pad), lambda: (0, 0)),
            pl.BlockSpec((1, P_pad), lambda: (0, 0)),
            pl.BlockSpec((n_parts, N, k_pad), lambda: (0, 0, 0)),
        ],
        out_specs=pl.BlockSpec((1, P_pad), lambda: (0, 0)),
        grid=(),
    )(r32, w, b, agg_parts)


def kernel(r_s, weight_W, bias, edge_index):
    N, F = r_s.shape
    P = weight_W.shape[1]
    E = edge_index.shape[1]
    k = P // 10

    P_pad = ((P + 127) // 128) * 128
    k_pad = min(((max(k, 1) + 127) // 128) * 128, P_pad)

    devs = [d for d in jax.devices() if d.platform == "tpu"]
    D = 2 if len(devs) >= 2 else 1

    E_TILE = 512
    n_steps = pl.cdiv(pl.cdiv(E, E_TILE), D)
    E_pad = D * n_steps * E_TILE

    r32 = r_s.astype(jnp.float32)
    w = jnp.zeros((F, P_pad), jnp.float32).at[:, :P].set(
        weight_W.astype(jnp.float32))
    b = jnp.zeros((1, P_pad), jnp.float32).at[:, :P].set(
        bias.astype(jnp.float32)[None, :])
    wk = w[:, :k_pad]
    row = jnp.full((E_pad,), -1, jnp.int32).at[:E].set(
        edge_index[0].astype(jnp.int32)).reshape(D, n_steps * E_TILE)
    col = jnp.full((E_pad,), -1, jnp.int32).at[:E].set(
        edge_index[1].astype(jnp.int32)).reshape(D, n_steps * E_TILE)

    edge_call = functools.partial(_edge_call, N=N, F=F, k_pad=k_pad,
                                  E_TILE=E_TILE, n_steps=n_steps)
    fin_call = functools.partial(_finalize_call, N=N, F=F, P_pad=P_pad,
                                 k=k, k_pad=k_pad)

    if D == 1:
        agg = edge_call(r32, wk, row, col)
        out = fin_call(r32, w, b, agg[None])
        return out[0, :P]

    mesh = Mesh(np.array(devs[:D]), ("x",))

    def _sharded(r32, wk, w, b, row_sh, col_sh):
        agg = edge_call(r32, wk, row_sh, col_sh)
        agg_all = lax.all_gather(agg, "x")
        out = fin_call(r32, w, b, agg_all)
        return out

    out = _shard_map(
        _sharded, mesh=mesh,
        in_specs=(PSpec(None, None), PSpec(None, None),
                  PSpec(None, None), PSpec(None, None),
                  PSpec("x", None), PSpec("x", None)),
        out_specs=PSpec("x", None),
        check_rep=False,
    )(r32, wk, w, b, row, col)
    return out[0, :P]


# speedup vs baseline: 1.9735x; 1.0370x over previous
import functools

import numpy as np

import jax
import jax.numpy as jnp
from jax import lax
from jax.experimental import pallas as pl
from jax.experimental.pallas import tpu as pltpu
from jax.sharding import Mesh, PartitionSpec as PSpec

try:
    from jax.experimental.shard_map import shard_map as _shard_map
except ImportError:
    _shard_map = jax.shard_map


def _edge_kernel(r_ref, wk_ref, row_ref, col_ref, out_ref, v_ref, agg_ref,
                 *, n_steps):
    e = pl.program_id(0)
    n_nodes = r_ref.shape[0]
    e_tile = row_ref.shape[1]

    @pl.when(e == 0)
    def _init():
        v_ref[...] = jnp.dot(r_ref[...], wk_ref[...],
                             preferred_element_type=jnp.float32
                             ).astype(jnp.bfloat16)
        agg_ref[...] = jnp.zeros_like(agg_ref)

    iota = lax.broadcasted_iota(jnp.int32, (n_nodes, e_tile), 0)
    row_oh = (iota == row_ref[...]).astype(jnp.bfloat16)
    col_oh = (iota == col_ref[...]).astype(jnp.bfloat16)

    e_vals = lax.dot_general(col_oh, v_ref[...],
                             dimension_numbers=(((0,), (0,)), ((), ())),
                             preferred_element_type=jnp.float32
                             ).astype(jnp.bfloat16)
    agg_ref[...] += lax.dot_general(row_oh, e_vals,
                                    dimension_numbers=(((1,), (0,)), ((), ())),
                                    preferred_element_type=jnp.float32)

    @pl.when(e == n_steps - 1)
    def _emit():
        out_ref[...] = agg_ref[...]


def _finalize_kernel(r_ref, w_ref, b_ref, agg_ref, out_ref, *, k, k_pad):
    n_nodes = r_ref.shape[0]
    n_parts = agg_ref.shape[0]
    v = jnp.dot(r_ref[...], w_ref[...],
                preferred_element_type=jnp.float32)
    agg = agg_ref[0]
    for i in range(1, n_parts):
        agg = agg + agg_ref[i]
    lane = lax.broadcasted_iota(jnp.int32, (n_nodes, k_pad), 1)
    head = jnp.where(lane < k, agg, v[:, 0:k_pad])
    if k_pad < v.shape[1]:
        combined = jnp.concatenate([head, v[:, k_pad:]], axis=1)
    else:
        combined = head
    combined = combined + b_ref[...]
    i_s = jnp.maximum(combined, jnp.float32(0.0))
    out_ref[...] = jnp.mean(i_s, axis=0, keepdims=True)


def _edge_call(r32, wk, row_sh, col_sh, *, N, F, k_pad, E_TILE, n_steps):
    return pl.pallas_call(
        functools.partial(_edge_kernel, n_steps=n_steps),
        out_shape=jax.ShapeDtypeStruct((N, k_pad), jnp.float32),
        grid_spec=pltpu.PrefetchScalarGridSpec(
            num_scalar_prefetch=0,
            grid=(n_steps,),
            in_specs=[
                pl.BlockSpec((N, F), lambda e: (0, 0)),
                pl.BlockSpec((F, k_pad), lambda e: (0, 0)),
                pl.BlockSpec((1, E_TILE), lambda e: (0, e)),
                pl.BlockSpec((1, E_TILE), lambda e: (0, e)),
            ],
            out_specs=pl.BlockSpec((N, k_pad), lambda e: (0, 0)),
            scratch_shapes=[
                pltpu.VMEM((N, k_pad), jnp.bfloat16),
                pltpu.VMEM((N, k_pad), jnp.float32),
            ],
        ),
        compiler_params=pltpu.CompilerParams(
            dimension_semantics=("arbitrary",),
        ),
    )(r32, wk, row_sh, col_sh)


def _finalize_call(r32, w, b, agg_parts, *, N, F, P_pad, k, k_pad):
    n_parts = agg_parts.shape[0]
    return pl.pallas_call(
        functools.partial(_finalize_kernel, k=k, k_pad=k_pad),
        out_shape=jax.ShapeDtypeStruct((1, P_pad), jnp.float32),
        in_specs=[
            pl.BlockSpec((N, F), lambda: (0, 0)),
            pl.BlockSpec((F, P_pad), lambda: (0, 0)),
            pl.BlockSpec((1, P_pad), lambda: (0, 0)),
            pl.BlockSpec((n_parts, N, k_pad), lambda: (0, 0, 0)),
        ],
        out_specs=pl.BlockSpec((1, P_pad), lambda: (0, 0)),
        grid=(),
    )(r32, w, b, agg_parts)


def kernel(r_s, weight_W, bias, edge_index):
    N, F = r_s.shape
    P = weight_W.shape[1]
    E = edge_index.shape[1]
    k = P // 10

    P_pad = ((P + 127) // 128) * 128
    k_pad = min(((max(k, 1) + 127) // 128) * 128, P_pad)

    devs = [d for d in jax.devices() if d.platform == "tpu"]
    D = 2 if len(devs) >= 2 else 1

    E_TILE = 1024
    n_steps = pl.cdiv(pl.cdiv(E, E_TILE), D)
    E_pad = D * n_steps * E_TILE

    r32 = r_s.astype(jnp.float32)
    w = jnp.zeros((F, P_pad), jnp.float32).at[:, :P].set(
        weight_W.astype(jnp.float32))
    b = jnp.zeros((1, P_pad), jnp.float32).at[:, :P].set(
        bias.astype(jnp.float32)[None, :])
    wk = w[:, :k_pad]
    row = jnp.full((E_pad,), -1, jnp.int32).at[:E].set(
        edge_index[0].astype(jnp.int32)).reshape(D, n_steps * E_TILE)
    col = jnp.full((E_pad,), -1, jnp.int32).at[:E].set(
        edge_index[1].astype(jnp.int32)).reshape(D, n_steps * E_TILE)

    edge_call = functools.partial(_edge_call, N=N, F=F, k_pad=k_pad,
                                  E_TILE=E_TILE, n_steps=n_steps)
    fin_call = functools.partial(_finalize_call, N=N, F=F, P_pad=P_pad,
                                 k=k, k_pad=k_pad)

    if D == 1:
        agg = edge_call(r32, wk, row, col)
        out = fin_call(r32, w, b, agg[None])
        return out[0, :P]

    mesh = Mesh(np.array(devs[:D]), ("x",))

    def _sharded(r32, wk, w, b, row_sh, col_sh):
        agg = edge_call(r32, wk, row_sh, col_sh)
        agg_all = lax.all_gather(agg, "x")
        out = fin_call(r32, w, b, agg_all)
        return out

    out = _shard_map(
        _sharded, mesh=mesh,
        in_specs=(PSpec(None, None), PSpec(None, None),
                  PSpec(None, None), PSpec(None, None),
                  PSpec("x", None), PSpec("x", None)),
        out_specs=PSpec("x", None),
        check_rep=False,
    )(r32, wk, w, b, row, col)
    return out[0, :P]


# speedup vs baseline: 3.1026x; 1.5721x over previous
import functools

import numpy as np

import jax
import jax.numpy as jnp
from jax import lax
from jax.experimental import pallas as pl
from jax.experimental.pallas import tpu as pltpu
from jax.sharding import Mesh, PartitionSpec as PSpec

try:
    from jax.experimental.shard_map import shard_map as _shard_map
except ImportError:
    _shard_map = jax.shard_map


def _edge_kernel(r_ref, wk_ref, row_ref, col_ref, out_ref, v_ref, agg_ref,
                 *, n_steps):
    e = pl.program_id(0)
    n_nodes = r_ref.shape[0]
    e_tile = row_ref.shape[1]

    @pl.when(e == 0)
    def _init():
        v_ref[...] = jnp.dot(r_ref[...], wk_ref[...],
                             preferred_element_type=jnp.float32
                             ).astype(jnp.bfloat16)
        agg_ref[...] = jnp.zeros_like(agg_ref)

    n_sub = 2
    sub = e_tile // n_sub
    parts = []
    for s in range(n_sub):
        row_s = row_ref[:, s * sub:(s + 1) * sub]
        col_s = col_ref[s * sub:(s + 1) * sub, :]
        row_iota = lax.broadcasted_iota(jnp.int32, (n_nodes, sub), 0)
        row_oh = (row_iota == row_s).astype(jnp.bfloat16)
        col_iota = lax.broadcasted_iota(jnp.int32, (sub, n_nodes), 1)
        col_oh_t = (col_iota == col_s).astype(jnp.bfloat16)

        e_vals = lax.dot_general(col_oh_t, v_ref[...],
                                 dimension_numbers=(((1,), (0,)), ((), ())),
                                 preferred_element_type=jnp.float32
                                 ).astype(jnp.bfloat16)
        parts.append(lax.dot_general(row_oh, e_vals,
                                     dimension_numbers=(((1,), (0,)), ((), ())),
                                     preferred_element_type=jnp.float32))
    acc = parts[0]
    for p in parts[1:]:
        acc = acc + p
    agg_ref[...] += acc

    @pl.when(e == n_steps - 1)
    def _emit():
        out_ref[...] = agg_ref[...]


def _finalize_kernel(r_ref, w_ref, b_ref, agg_ref, out_ref, *, k, k_pad):
    n_nodes = r_ref.shape[0]
    n_parts = agg_ref.shape[0]
    v = jnp.dot(r_ref[...], w_ref[...],
                preferred_element_type=jnp.float32)
    agg = agg_ref[0]
    for i in range(1, n_parts):
        agg = agg + agg_ref[i]
    lane = lax.broadcasted_iota(jnp.int32, (n_nodes, k_pad), 1)
    head = jnp.where(lane < k, agg, v[:, 0:k_pad])
    if k_pad < v.shape[1]:
        combined = jnp.concatenate([head, v[:, k_pad:]], axis=1)
    else:
        combined = head
    combined = combined + b_ref[...]
    i_s = jnp.maximum(combined, jnp.float32(0.0))
    out_ref[...] = jnp.mean(i_s, axis=0, keepdims=True)


def _edge_call(r32, wk, row_sh, col_sh, *, N, F, k_pad, E_TILE, n_steps):
    return pl.pallas_call(
        functools.partial(_edge_kernel, n_steps=n_steps),
        out_shape=jax.ShapeDtypeStruct((N, k_pad), jnp.float32),
        grid_spec=pltpu.PrefetchScalarGridSpec(
            num_scalar_prefetch=0,
            grid=(n_steps,),
            in_specs=[
                pl.BlockSpec((N, F), lambda e: (0, 0)),
                pl.BlockSpec((F, k_pad), lambda e: (0, 0)),
                pl.BlockSpec((1, E_TILE), lambda e: (0, e)),
                pl.BlockSpec((E_TILE, 1), lambda e: (e, 0)),
            ],
            out_specs=pl.BlockSpec((N, k_pad), lambda e: (0, 0)),
            scratch_shapes=[
                pltpu.VMEM((N, k_pad), jnp.bfloat16),
                pltpu.VMEM((N, k_pad), jnp.float32),
            ],
        ),
        compiler_params=pltpu.CompilerParams(
            dimension_semantics=("arbitrary",),
        ),
    )(r32, wk, row_sh, col_sh)


def _finalize_call(r32, w, b, agg_parts, *, N, F, P_pad, k, k_pad):
    n_parts = agg_parts.shape[0]
    return pl.pallas_call(
        functools.partial(_finalize_kernel, k=k, k_pad=k_pad),
        out_shape=jax.ShapeDtypeStruct((1, P_pad), jnp.float32),
        in_specs=[
            pl.BlockSpec((N, F), lambda: (0, 0)),
            pl.BlockSpec((F, P_pad), lambda: (0, 0)),
            pl.BlockSpec((1, P_pad), lambda: (0, 0)),
            pl.BlockSpec((n_parts, N, k_pad), lambda: (0, 0, 0)),
        ],
        out_specs=pl.BlockSpec((1, P_pad), lambda: (0, 0)),
        grid=(),
    )(r32, w, b, agg_parts)


def kernel(r_s, weight_W, bias, edge_index):
    N, F = r_s.shape
    P = weight_W.shape[1]
    E = edge_index.shape[1]
    k = P // 10

    P_pad = ((P + 127) // 128) * 128
    k_pad = min(((max(k, 1) + 127) // 128) * 128, P_pad)

    devs = [d for d in jax.devices() if d.platform == "tpu"]
    D = 2 if len(devs) >= 2 else 1

    E_TILE = 1024
    n_steps = pl.cdiv(pl.cdiv(E, E_TILE), D)
    E_pad = D * n_steps * E_TILE

    r32 = r_s.astype(jnp.float32)
    w = jnp.zeros((F, P_pad), jnp.float32).at[:, :P].set(
        weight_W.astype(jnp.float32))
    b = jnp.zeros((1, P_pad), jnp.float32).at[:, :P].set(
        bias.astype(jnp.float32)[None, :])
    wk = w[:, :k_pad]
    row = jnp.full((E_pad,), -1, jnp.int32).at[:E].set(
        edge_index[0].astype(jnp.int32)).reshape(D, n_steps * E_TILE)
    col = jnp.full((E_pad, 1), -1, jnp.int32).at[:E, 0].set(
        edge_index[1].astype(jnp.int32))

    edge_call = functools.partial(_edge_call, N=N, F=F, k_pad=k_pad,
                                  E_TILE=E_TILE, n_steps=n_steps)
    fin_call = functools.partial(_finalize_call, N=N, F=F, P_pad=P_pad,
                                 k=k, k_pad=k_pad)

    if D == 1:
        agg = edge_call(r32, wk, row, col)
        out = fin_call(r32, w, b, agg[None])
        return out[0, :P]

    mesh = Mesh(np.array(devs[:D]), ("x",))

    def _sharded(r32, wk, w, b, row_sh, col_sh):
        agg = edge_call(r32, wk, row_sh, col_sh)
        agg_all = lax.all_gather(agg, "x")
        out = fin_call(r32, w, b, agg_all)
        return out

    out = _shard_map(
        _sharded, mesh=mesh,
        in_specs=(PSpec(None, None), PSpec(None, None),
                  PSpec(None, None), PSpec(None, None),
                  PSpec("x", None), PSpec("x", None)),
        out_specs=PSpec("x", None),
        check_rep=False,
    )(r32, wk, w, b, row, col)
    return out[0, :P]


# speedup vs baseline: 4.1160x; 1.3266x over previous
import functools

import numpy as np

import jax
import jax.numpy as jnp
from jax import lax
from jax.experimental import pallas as pl
from jax.experimental.pallas import tpu as pltpu
from jax.sharding import Mesh, PartitionSpec as PSpec

try:
    from jax.experimental.shard_map import shard_map as _shard_map
except ImportError:
    _shard_map = jax.shard_map


def _edge_kernel(r_ref, wk_ref, row_ref, rowc_ref, col_ref, out_ref,
                 v3_ref, agg_ref, *, n_steps, n_sub, nh, nl, k_pad):
    e = pl.program_id(0)
    e_tile = row_ref.shape[1]
    sub = e_tile // n_sub

    @pl.when(e == 0)
    def _init():
        v128 = jnp.dot(r_ref[...], wk_ref[...],
                       preferred_element_type=jnp.float32)
        for h in range(nh):
            v3_ref[:, h * k_pad:(h + 1) * k_pad] = (
                v128[h * nl:(h + 1) * nl, :].astype(jnp.bfloat16))
        agg_ref[...] = jnp.zeros_like(agg_ref)

    shift = jnp.int32(nl.bit_length() - 1)
    for s in range(n_sub):
        colv = col_ref[s * sub:(s + 1) * sub, :]
        clo = colv & jnp.int32(nl - 1)
        lo_iota = lax.broadcasted_iota(jnp.int32, (sub, nl), 1)
        lo_oh = (lo_iota == clo).astype(jnp.bfloat16)
        g = lax.dot_general(lo_oh, v3_ref[...],
                            dimension_numbers=(((1,), (0,)), ((), ())),
                            preferred_element_type=jnp.float32)
        chi_b = jnp.broadcast_to(colv, (sub, k_pad)) >> shift
        ev = jnp.zeros((sub, k_pad), jnp.float32)
        for h in range(nh):
            ev = ev + jnp.where(chi_b == h, g[:, h * k_pad:(h + 1) * k_pad],
                                jnp.float32(0.0))
        e_vals = ev.astype(jnp.bfloat16)

        rowv = rowc_ref[s * sub:(s + 1) * sub, :]
        rhi_b = jnp.broadcast_to(rowv, (sub, k_pad)) >> shift
        xs = [jnp.where(rhi_b == h, e_vals, jnp.bfloat16(0.0))
              for h in range(nh)]
        x = jnp.concatenate(xs, axis=1)

        rlo = row_ref[:, s * sub:(s + 1) * sub] & jnp.int32(nl - 1)
        rlo_iota = lax.broadcasted_iota(jnp.int32, (nl, sub), 0)
        rlo_oh = (rlo_iota == rlo).astype(jnp.bfloat16)
        agg_ref[...] += lax.dot_general(
            rlo_oh, x, dimension_numbers=(((1,), (0,)), ((), ())),
            preferred_element_type=jnp.float32)

    @pl.when(e == n_steps - 1)
    def _emit():
        for h in range(nh):
            out_ref[h * nl:(h + 1) * nl, :] = (
                agg_ref[:, h * k_pad:(h + 1) * k_pad])


def _finalize_kernel(r_ref, w_ref, b_ref, agg_ref, out_ref, *, k, k_pad):
    n_nodes = r_ref.shape[0]
    n_parts = agg_ref.shape[0]
    v = jnp.dot(r_ref[...], w_ref[...],
                preferred_element_type=jnp.float32)
    agg = agg_ref[0]
    for i in range(1, n_parts):
        agg = agg + agg_ref[i]
    lane = lax.broadcasted_iota(jnp.int32, (n_nodes, k_pad), 1)
    head = jnp.where(lane < k, agg, v[:, 0:k_pad])
    if k_pad < v.shape[1]:
        combined = jnp.concatenate([head, v[:, k_pad:]], axis=1)
    else:
        combined = head
    combined = combined + b_ref[...]
    i_s = jnp.maximum(combined, jnp.float32(0.0))
    out_ref[...] = jnp.mean(i_s, axis=0, keepdims=True)


def _edge_call(r32, wk, row, rowc, col, *, N, F, k_pad, E_TILE, n_steps,
               n_sub, nh, nl):
    body = functools.partial(_edge_kernel, n_steps=n_steps, n_sub=n_sub,
                             nh=nh, nl=nl, k_pad=k_pad)
    return pl.pallas_call(
        body,
        out_shape=jax.ShapeDtypeStruct((N, k_pad), jnp.float32),
        grid_spec=pltpu.PrefetchScalarGridSpec(
            num_scalar_prefetch=0,
            grid=(n_steps,),
            in_specs=[
                pl.BlockSpec((N, F), lambda e: (0, 0)),
                pl.BlockSpec((F, k_pad), lambda e: (0, 0)),
                pl.BlockSpec((1, E_TILE), lambda e: (0, e)),
                pl.BlockSpec((E_TILE, 1), lambda e: (e, 0)),
                pl.BlockSpec((E_TILE, 1), lambda e: (e, 0)),
            ],
            out_specs=pl.BlockSpec((N, k_pad), lambda e: (0, 0)),
            scratch_shapes=[
                pltpu.VMEM((nl, nh * k_pad), jnp.bfloat16),
                pltpu.VMEM((nl, nh * k_pad), jnp.float32),
            ],
        ),
        compiler_params=pltpu.CompilerParams(
            dimension_semantics=("arbitrary",),
        ),
    )(r32, wk, row, rowc, col)


def _finalize_call(r32, w, b, agg_parts, *, N, F, P_pad, k, k_pad):
    n_parts = agg_parts.shape[0]
    return pl.pallas_call(
        functools.partial(_finalize_kernel, k=k, k_pad=k_pad),
        out_shape=jax.ShapeDtypeStruct((1, P_pad), jnp.float32),
        in_specs=[
            pl.BlockSpec((N, F), lambda: (0, 0)),
            pl.BlockSpec((F, P_pad), lambda: (0, 0)),
            pl.BlockSpec((1, P_pad), lambda: (0, 0)),
            pl.BlockSpec((n_parts, N, k_pad), lambda: (0, 0, 0)),
        ],
        out_specs=pl.BlockSpec((1, P_pad), lambda: (0, 0)),
        grid=(),
    )(r32, w, b, agg_parts)


def kernel(r_s, weight_W, bias, edge_index):
    N, F = r_s.shape
    P = weight_W.shape[1]
    E = edge_index.shape[1]
    k = P // 10

    P_pad = ((P + 127) // 128) * 128
    k_pad = min(((max(k, 1) + 127) // 128) * 128, P_pad)

    devs = [d for d in jax.devices() if d.platform == "tpu"]
    D = 2 if len(devs) >= 2 else 1

    nl = 512
    while nl >= N and nl > 8:
        nl //= 2
    nh = N // nl

    E_TILE = 1024
    n_sub = 2
    n_steps = pl.cdiv(pl.cdiv(E, E_TILE), D)
    E_pad = D * n_steps * E_TILE

    r32 = r_s.astype(jnp.float32)
    w = jnp.zeros((F, P_pad), jnp.float32).at[:, :P].set(
        weight_W.astype(jnp.float32))
    b = jnp.zeros((1, P_pad), jnp.float32).at[:, :P].set(
        bias.astype(jnp.float32)[None, :])
    wk = w[:, :k_pad]
    row_flat = jnp.full((E_pad,), -1, jnp.int32).at[:E].set(
        edge_index[0].astype(jnp.int32))
    row = row_flat.reshape(D, n_steps * E_TILE)
    rowc = row_flat[:, None]
    col = jnp.full((E_pad, 1), -1, jnp.int32).at[:E, 0].set(
        edge_index[1].astype(jnp.int32))

    edge_call = functools.partial(_edge_call, N=N, F=F, k_pad=k_pad,
                                  E_TILE=E_TILE, n_steps=n_steps,
                                  n_sub=n_sub, nh=nh, nl=nl)
    fin_call = functools.partial(_finalize_call, N=N, F=F, P_pad=P_pad,
                                 k=k, k_pad=k_pad)

    if D == 1:
        agg = edge_call(r32, wk, row, rowc, col)
        out = fin_call(r32, w, b, agg[None])
        return out[0, :P]

    mesh = Mesh(np.array(devs[:D]), ("x",))

    def _sharded(r32, wk, w, b, row_sh, rowc_sh, col_sh):
        agg = edge_call(r32, wk, row_sh, rowc_sh, col_sh)
        agg_all = lax.all_gather(agg, "x")
        out = fin_call(r32, w, b, agg_all)
        return out

    out = _shard_map(
        _sharded, mesh=mesh,
        in_specs=(PSpec(None, None), PSpec(None, None),
                  PSpec(None, None), PSpec(None, None),
                  PSpec("x", None), PSpec("x", None), PSpec("x", None)),
        out_specs=PSpec("x", None),
        check_rep=False,
    )(r32, wk, w, b, row, rowc, col)
    return out[0, :P]


# speedup vs baseline: 4.2610x; 1.0352x over previous
import functools

import numpy as np

import jax
import jax.numpy as jnp
from jax import lax
from jax.experimental import pallas as pl
from jax.experimental.pallas import tpu as pltpu
from jax.sharding import Mesh, PartitionSpec as PSpec

try:
    from jax.experimental.shard_map import shard_map as _shard_map
except ImportError:
    _shard_map = jax.shard_map


def _edge_kernel(r_ref, wk_ref, row_ref, rowc_ref, col_ref, out_ref,
                 v3_ref, agg_ref, *, n_steps, n_sub, nh, nl, k_pad):
    e = pl.program_id(0)
    e_tile = row_ref.shape[1]
    sub = e_tile // n_sub

    @pl.when(e == 0)
    def _init():
        v128 = jnp.dot(r_ref[...], wk_ref[...],
                       preferred_element_type=jnp.float32)
        for h in range(nh):
            v3_ref[:, h * k_pad:(h + 1) * k_pad] = (
                v128[h * nl:(h + 1) * nl, :].astype(jnp.bfloat16))
        agg_ref[...] = jnp.zeros_like(agg_ref)

    shift = jnp.int32(nl.bit_length() - 1)
    for s in range(n_sub):
        colv = col_ref[s * sub:(s + 1) * sub, :]
        clo = colv & jnp.int32(nl - 1)
        lo_iota = lax.broadcasted_iota(jnp.int32, (sub, nl), 1)
        lo_oh = (lo_iota == clo).astype(jnp.bfloat16)
        g = lax.dot_general(lo_oh, v3_ref[...],
                            dimension_numbers=(((1,), (0,)), ((), ())),
                            preferred_element_type=jnp.float32)
        chi_b = jnp.broadcast_to(colv, (sub, k_pad)) >> shift
        ev = jnp.zeros((sub, k_pad), jnp.float32)
        for h in range(nh):
            ev = ev + jnp.where(chi_b == h, g[:, h * k_pad:(h + 1) * k_pad],
                                jnp.float32(0.0))
        e_vals = ev.astype(jnp.bfloat16)

        rowv = rowc_ref[s * sub:(s + 1) * sub, :]
        rhi_b = jnp.broadcast_to(rowv, (sub, k_pad)) >> shift
        xs = [jnp.where(rhi_b == h, e_vals, jnp.bfloat16(0.0))
              for h in range(nh)]
        x = jnp.concatenate(xs, axis=1)

        rlo = row_ref[:, s * sub:(s + 1) * sub] & jnp.int32(nl - 1)
        rlo_iota = lax.broadcasted_iota(jnp.int32, (nl, sub), 0)
        rlo_oh = (rlo_iota == rlo).astype(jnp.bfloat16)
        agg_ref[...] += lax.dot_general(
            rlo_oh, x, dimension_numbers=(((1,), (0,)), ((), ())),
            preferred_element_type=jnp.float32)

    @pl.when(e == n_steps - 1)
    def _emit():
        for h in range(nh):
            out_ref[h * nl:(h + 1) * nl, :] = (
                agg_ref[:, h * k_pad:(h + 1) * k_pad])


def _finalize_kernel(r_ref, w_ref, b_ref, agg_ref, out_ref, *, k, k_pad):
    n_nodes = r_ref.shape[0]
    n_parts = agg_ref.shape[0]
    v = jnp.dot(r_ref[...], w_ref[...],
                preferred_element_type=jnp.float32)
    agg = agg_ref[0]
    for i in range(1, n_parts):
        agg = agg + agg_ref[i]
    lane = lax.broadcasted_iota(jnp.int32, (n_nodes, k_pad), 1)
    head = jnp.where(lane < k, agg, v[:, 0:k_pad])
    if k_pad < v.shape[1]:
        combined = jnp.concatenate([head, v[:, k_pad:]], axis=1)
    else:
        combined = head
    combined = combined + b_ref[...]
    i_s = jnp.maximum(combined, jnp.float32(0.0))
    out_ref[...] = jnp.mean(i_s, axis=0, keepdims=True)


def _edge_call(r32, wk, row, rowc, col, *, N, F, k_pad, E_TILE, n_steps,
               n_sub, nh, nl):
    body = functools.partial(_edge_kernel, n_steps=n_steps, n_sub=n_sub,
                             nh=nh, nl=nl, k_pad=k_pad)
    return pl.pallas_call(
        body,
        out_shape=jax.ShapeDtypeStruct((N, k_pad), jnp.float32),
        grid_spec=pltpu.PrefetchScalarGridSpec(
            num_scalar_prefetch=0,
            grid=(n_steps,),
            in_specs=[
                pl.BlockSpec((N, F), lambda e: (0, 0)),
                pl.BlockSpec((F, k_pad), lambda e: (0, 0)),
                pl.BlockSpec((1, E_TILE), lambda e: (0, e)),
                pl.BlockSpec((E_TILE, 1), lambda e: (e, 0)),
                pl.BlockSpec((E_TILE, 1), lambda e: (e, 0)),
            ],
            out_specs=pl.BlockSpec((N, k_pad), lambda e: (0, 0)),
            scratch_shapes=[
                pltpu.VMEM((nl, nh * k_pad), jnp.bfloat16),
                pltpu.VMEM((nl, nh * k_pad), jnp.float32),
            ],
        ),
        compiler_params=pltpu.CompilerParams(
            dimension_semantics=("arbitrary",),
        ),
    )(r32, wk, row, rowc, col)


def _finalize_call(r32, w, b, agg_parts, *, N, F, P_pad, k, k_pad):
    n_parts = agg_parts.shape[0]
    return pl.pallas_call(
        functools.partial(_finalize_kernel, k=k, k_pad=k_pad),
        out_shape=jax.ShapeDtypeStruct((1, P_pad), jnp.float32),
        in_specs=[
            pl.BlockSpec((N, F), lambda: (0, 0)),
            pl.BlockSpec((F, P_pad), lambda: (0, 0)),
            pl.BlockSpec((1, P_pad), lambda: (0, 0)),
            pl.BlockSpec((n_parts, N, k_pad), lambda: (0, 0, 0)),
        ],
        out_specs=pl.BlockSpec((1, P_pad), lambda: (0, 0)),
        grid=(),
    )(r32, w, b, agg_parts)


def kernel(r_s, weight_W, bias, edge_index):
    N, F = r_s.shape
    P = weight_W.shape[1]
    E = edge_index.shape[1]
    k = P // 10

    P_pad = ((P + 127) // 128) * 128
    k_pad = min(((max(k, 1) + 127) // 128) * 128, P_pad)

    devs = [d for d in jax.devices() if d.platform == "tpu"]
    D = 2 if len(devs) >= 2 else 1

    nl = 1024
    while nl >= N and nl > 8:
        nl //= 2
    nh = N // nl

    E_TILE = 1024
    n_sub = 2
    n_steps = pl.cdiv(pl.cdiv(E, E_TILE), D)
    E_pad = D * n_steps * E_TILE

    r32 = r_s.astype(jnp.float32)
    w = jnp.zeros((F, P_pad), jnp.float32).at[:, :P].set(
        weight_W.astype(jnp.float32))
    b = jnp.zeros((1, P_pad), jnp.float32).at[:, :P].set(
        bias.astype(jnp.float32)[None, :])
    wk = w[:, :k_pad]
    row_flat = jnp.full((E_pad,), -1, jnp.int32).at[:E].set(
        edge_index[0].astype(jnp.int32))
    row = row_flat.reshape(D, n_steps * E_TILE)
    rowc = row_flat[:, None]
    col = jnp.full((E_pad, 1), -1, jnp.int32).at[:E, 0].set(
        edge_index[1].astype(jnp.int32))

    edge_call = functools.partial(_edge_call, N=N, F=F, k_pad=k_pad,
                                  E_TILE=E_TILE, n_steps=n_steps,
                                  n_sub=n_sub, nh=nh, nl=nl)
    fin_call = functools.partial(_finalize_call, N=N, F=F, P_pad=P_pad,
                                 k=k, k_pad=k_pad)

    if D == 1:
        agg = edge_call(r32, wk, row, rowc, col)
        out = fin_call(r32, w, b, agg[None])
        return out[0, :P]

    mesh = Mesh(np.array(devs[:D]), ("x",))

    def _sharded(r32, wk, w, b, row_sh, rowc_sh, col_sh):
        agg = edge_call(r32, wk, row_sh, rowc_sh, col_sh)
        agg_all = lax.all_gather(agg, "x")
        out = fin_call(r32, w, b, agg_all)
        return out

    out = _shard_map(
        _sharded, mesh=mesh,
        in_specs=(PSpec(None, None), PSpec(None, None),
                  PSpec(None, None), PSpec(None, None),
                  PSpec("x", None), PSpec("x", None), PSpec("x", None)),
        out_specs=PSpec("x", None),
        check_rep=False,
    )(r32, wk, w, b, row, rowc, col)
    return out[0, :P]


# speedup vs baseline: 4.4665x; 1.0482x over previous
import functools

import numpy as np

import jax
import jax.numpy as jnp
from jax import lax
from jax.experimental import pallas as pl
from jax.experimental.pallas import tpu as pltpu
from jax.sharding import Mesh, PartitionSpec as PSpec

try:
    from jax.experimental.shard_map import shard_map as _shard_map
except ImportError:
    _shard_map = jax.shard_map


def _edge_kernel(r_ref, wk_ref, row_ref, rowc_ref, col_ref, out_ref,
                 v3_ref, agg_ref, *, n_steps, n_sub, nh, nl, k_pad):
    e = pl.program_id(0)
    e_tile = row_ref.shape[1]
    sub = e_tile // n_sub

    @pl.when(e == 0)
    def _init():
        v128 = jnp.dot(r_ref[...], wk_ref[...],
                       preferred_element_type=jnp.float32)
        for h in range(nh):
            v3_ref[:, h * k_pad:(h + 1) * k_pad] = (
                v128[h * nl:(h + 1) * nl, :].astype(jnp.bfloat16))
        agg_ref[...] = jnp.zeros_like(agg_ref)

    shift = jnp.int32(nl.bit_length() - 1)
    for s in range(n_sub):
        colv = col_ref[s * sub:(s + 1) * sub, :]
        clo = colv & jnp.int32(nl - 1)
        lo_iota = lax.broadcasted_iota(jnp.int32, (sub, nl), 1)
        lo_oh = (lo_iota == clo).astype(jnp.bfloat16)
        g = lax.dot_general(lo_oh, v3_ref[...],
                            dimension_numbers=(((1,), (0,)), ((), ())),
                            preferred_element_type=jnp.float32)
        chi_b = jnp.broadcast_to(colv, (sub, k_pad)) >> shift
        ev = jnp.zeros((sub, k_pad), jnp.float32)
        for h in range(nh):
            ev = ev + jnp.where(chi_b == h, g[:, h * k_pad:(h + 1) * k_pad],
                                jnp.float32(0.0))
        e_vals = ev.astype(jnp.bfloat16)

        rowv = rowc_ref[s * sub:(s + 1) * sub, :]
        rhi_b = jnp.broadcast_to(rowv, (sub, k_pad)) >> shift
        xs = [jnp.where(rhi_b == h, e_vals, jnp.bfloat16(0.0))
              for h in range(nh)]
        x = jnp.concatenate(xs, axis=1)

        rlo = row_ref[:, s * sub:(s + 1) * sub] & jnp.int32(nl - 1)
        rlo_iota = lax.broadcasted_iota(jnp.int32, (nl, sub), 0)
        rlo_oh = (rlo_iota == rlo).astype(jnp.bfloat16)
        agg_ref[...] += lax.dot_general(
            rlo_oh, x, dimension_numbers=(((1,), (0,)), ((), ())),
            preferred_element_type=jnp.float32)

    @pl.when(e == n_steps - 1)
    def _emit():
        for h in range(nh):
            out_ref[h * nl:(h + 1) * nl, :] = (
                agg_ref[:, h * k_pad:(h + 1) * k_pad])


def _finalize_kernel(r_ref, w_ref, b_ref, agg_ref, out_ref, *, k, k_pad):
    n_nodes = r_ref.shape[0]
    n_parts = agg_ref.shape[0]
    v = jnp.dot(r_ref[...], w_ref[...],
                preferred_element_type=jnp.float32)
    agg = agg_ref[0]
    for i in range(1, n_parts):
        agg = agg + agg_ref[i]
    lane = lax.broadcasted_iota(jnp.int32, (n_nodes, k_pad), 1)
    head = jnp.where(lane < k, agg, v[:, 0:k_pad])
    if k_pad < v.shape[1]:
        combined = jnp.concatenate([head, v[:, k_pad:]], axis=1)
    else:
        combined = head
    combined = combined + b_ref[...]
    i_s = jnp.maximum(combined, jnp.float32(0.0))
    out_ref[...] = jnp.mean(i_s, axis=0, keepdims=True)


def _edge_call(r32, wk, row, rowc, col, *, N, F, k_pad, E_TILE, n_steps,
               n_sub, nh, nl):
    body = functools.partial(_edge_kernel, n_steps=n_steps, n_sub=n_sub,
                             nh=nh, nl=nl, k_pad=k_pad)
    return pl.pallas_call(
        body,
        out_shape=jax.ShapeDtypeStruct((N, k_pad), jnp.float32),
        grid_spec=pltpu.PrefetchScalarGridSpec(
            num_scalar_prefetch=0,
            grid=(n_steps,),
            in_specs=[
                pl.BlockSpec((N, F), lambda e: (0, 0)),
                pl.BlockSpec((F, k_pad), lambda e: (0, 0)),
                pl.BlockSpec((1, E_TILE), lambda e: (0, e)),
                pl.BlockSpec((E_TILE, 1), lambda e: (e, 0)),
                pl.BlockSpec((E_TILE, 1), lambda e: (e, 0)),
            ],
            out_specs=pl.BlockSpec((N, k_pad), lambda e: (0, 0)),
            scratch_shapes=[
                pltpu.VMEM((nl, nh * k_pad), jnp.bfloat16),
                pltpu.VMEM((nl, nh * k_pad), jnp.float32),
            ],
        ),
        compiler_params=pltpu.CompilerParams(
            dimension_semantics=("arbitrary",),
        ),
    )(r32, wk, row, rowc, col)


def _finalize_call(r32, w, b, agg_parts, *, N, F, P_pad, k, k_pad):
    n_parts = agg_parts.shape[0]
    return pl.pallas_call(
        functools.partial(_finalize_kernel, k=k, k_pad=k_pad),
        out_shape=jax.ShapeDtypeStruct((1, P_pad), jnp.float32),
        in_specs=[
            pl.BlockSpec((N, F), lambda: (0, 0)),
            pl.BlockSpec((F, P_pad), lambda: (0, 0)),
            pl.BlockSpec((1, P_pad), lambda: (0, 0)),
            pl.BlockSpec((n_parts, N, k_pad), lambda: (0, 0, 0)),
        ],
        out_specs=pl.BlockSpec((1, P_pad), lambda: (0, 0)),
        grid=(),
    )(r32, w, b, agg_parts)


def kernel(r_s, weight_W, bias, edge_index):
    N, F = r_s.shape
    P = weight_W.shape[1]
    E = edge_index.shape[1]
    k = P // 10

    P_pad = ((P + 127) // 128) * 128
    k_pad = min(((max(k, 1) + 127) // 128) * 128, P_pad)

    devs = [d for d in jax.devices() if d.platform == "tpu"]
    D = 2 if len(devs) >= 2 else 1

    nl = 1024
    while nl >= N and nl > 8:
        nl //= 2
    nh = N // nl

    E_TILE = 2048
    n_sub = 2
    n_steps = pl.cdiv(pl.cdiv(E, E_TILE), D)
    E_pad = D * n_steps * E_TILE

    r32 = r_s.astype(jnp.float32)
    w = jnp.zeros((F, P_pad), jnp.float32).at[:, :P].set(
        weight_W.astype(jnp.float32))
    b = jnp.zeros((1, P_pad), jnp.float32).at[:, :P].set(
        bias.astype(jnp.float32)[None, :])
    wk = w[:, :k_pad]
    row_flat = jnp.full((E_pad,), -1, jnp.int32).at[:E].set(
        edge_index[0].astype(jnp.int32))
    row = row_flat.reshape(D, n_steps * E_TILE)
    rowc = row_flat[:, None]
    col = jnp.full((E_pad, 1), -1, jnp.int32).at[:E, 0].set(
        edge_index[1].astype(jnp.int32))

    edge_call = functools.partial(_edge_call, N=N, F=F, k_pad=k_pad,
                                  E_TILE=E_TILE, n_steps=n_steps,
                                  n_sub=n_sub, nh=nh, nl=nl)
    fin_call = functools.partial(_finalize_call, N=N, F=F, P_pad=P_pad,
                                 k=k, k_pad=k_pad)

    if D == 1:
        agg = edge_call(r32, wk, row, rowc, col)
        out = fin_call(r32, w, b, agg[None])
        return out[0, :P]

    mesh = Mesh(np.array(devs[:D]), ("x",))

    def _sharded(r32, wk, w, b, row_sh, rowc_sh, col_sh):
        agg = edge_call(r32, wk, row_sh, rowc_sh, col_sh)
        agg_all = lax.all_gather(agg, "x")
        out = fin_call(r32, w, b, agg_all)
        return out

    out = _shard_map(
        _sharded, mesh=mesh,
        in_specs=(PSpec(None, None), PSpec(None, None),
                  PSpec(None, None), PSpec(None, None),
                  PSpec("x", None), PSpec("x", None), PSpec("x", None)),
        out_specs=PSpec("x", None),
        check_rep=False,
    )(r32, wk, w, b, row, rowc, col)
    return out[0, :P]


# speedup vs baseline: 7.0979x; 1.5892x over previous
import functools

import numpy as np

import jax
import jax.numpy as jnp
from jax import lax
from jax.experimental import pallas as pl
from jax.experimental.pallas import tpu as pltpu
from jax.sharding import Mesh, PartitionSpec as PSpec

try:
    from jax.experimental.shard_map import shard_map as _shard_map
except ImportError:
    _shard_map = jax.shard_map


def _edge_kernel(r_ref, wk_ref, row_ref, rowc_ref, col_ref, out_ref,
                 v1_ref, ev_ref, agg_ref, *, n_steps, n_sub, nh, nl, k_pad):
    e = pl.program_id(0)
    e_tile = row_ref.shape[1]
    sub = e_tile // n_sub

    @pl.when(e == 0)
    def _init():
        v128 = jnp.dot(r_ref[...], wk_ref[...],
                       preferred_element_type=jnp.float32)
        v1_ref[...] = v128[:, None, :]
        agg_ref[...] = jnp.zeros_like(agg_ref)

    for i in range(e_tile):
        ev_ref[i, :] = v1_ref[col_ref[i], 0, :]

    shift = jnp.int32(nl.bit_length() - 1)
    for s in range(n_sub):
        e_vals = ev_ref[s * sub:(s + 1) * sub, :].astype(jnp.bfloat16)

        rowv = rowc_ref[s * sub:(s + 1) * sub, :]
        rhi_b = jnp.broadcast_to(rowv, (sub, k_pad)) >> shift
        xs = [jnp.where(rhi_b == h, e_vals, jnp.bfloat16(0.0))
              for h in range(nh)]
        x = jnp.concatenate(xs, axis=1)

        rlo = row_ref[:, s * sub:(s + 1) * sub] & jnp.int32(nl - 1)
        rlo_iota = lax.broadcasted_iota(jnp.int32, (nl, sub), 0)
        rlo_oh = (rlo_iota == rlo).astype(jnp.bfloat16)
        agg_ref[...] += lax.dot_general(
            rlo_oh, x, dimension_numbers=(((1,), (0,)), ((), ())),
            preferred_element_type=jnp.float32)

    @pl.when(e == n_steps - 1)
    def _emit():
        for h in range(nh):
            out_ref[h * nl:(h + 1) * nl, :] = (
                agg_ref[:, h * k_pad:(h + 1) * k_pad])


def _finalize_kernel(r_ref, w_ref, b_ref, agg_ref, out_ref, *, k, k_pad):
    n_nodes = r_ref.shape[0]
    n_parts = agg_ref.shape[0]
    v = jnp.dot(r_ref[...], w_ref[...],
                preferred_element_type=jnp.float32)
    agg = agg_ref[0]
    for i in range(1, n_parts):
        agg = agg + agg_ref[i]
    lane = lax.broadcasted_iota(jnp.int32, (n_nodes, k_pad), 1)
    head = jnp.where(lane < k, agg, v[:, 0:k_pad])
    if k_pad < v.shape[1]:
        combined = jnp.concatenate([head, v[:, k_pad:]], axis=1)
    else:
        combined = head
    combined = combined + b_ref[...]
    i_s = jnp.maximum(combined, jnp.float32(0.0))
    out_ref[...] = jnp.mean(i_s, axis=0, keepdims=True)


def _edge_call(r32, wk, row, rowc, col, *, N, F, k_pad, E_TILE, n_steps,
               n_sub, nh, nl):
    body = functools.partial(_edge_kernel, n_steps=n_steps, n_sub=n_sub,
                             nh=nh, nl=nl, k_pad=k_pad)
    return pl.pallas_call(
        body,
        out_shape=jax.ShapeDtypeStruct((N, k_pad), jnp.float32),
        grid_spec=pltpu.PrefetchScalarGridSpec(
            num_scalar_prefetch=0,
            grid=(n_steps,),
            in_specs=[
                pl.BlockSpec((N, F), lambda e: (0, 0)),
                pl.BlockSpec((F, k_pad), lambda e: (0, 0)),
                pl.BlockSpec((1, E_TILE), lambda e: (0, e)),
                pl.BlockSpec((E_TILE, 1), lambda e: (e, 0)),
                pl.BlockSpec((E_TILE,), lambda e: (e,),
                             memory_space=pltpu.SMEM),
            ],
            out_specs=pl.BlockSpec((N, k_pad), lambda e: (0, 0)),
            scratch_shapes=[
                pltpu.VMEM((N, 1, k_pad), jnp.float32),
                pltpu.VMEM((E_TILE, k_pad), jnp.float32),
                pltpu.VMEM((nl, nh * k_pad), jnp.float32),
            ],
        ),
        compiler_params=pltpu.CompilerParams(
            dimension_semantics=("arbitrary",),
        ),
    )(r32, wk, row, rowc, col)


def _finalize_call(r32, w, b, agg_parts, *, N, F, P_pad, k, k_pad):
    n_parts = agg_parts.shape[0]
    return pl.pallas_call(
        functools.partial(_finalize_kernel, k=k, k_pad=k_pad),
        out_shape=jax.ShapeDtypeStruct((1, P_pad), jnp.float32),
        in_specs=[
            pl.BlockSpec((N, F), lambda: (0, 0)),
            pl.BlockSpec((F, P_pad), lambda: (0, 0)),
            pl.BlockSpec((1, P_pad), lambda: (0, 0)),
            pl.BlockSpec((n_parts, N, k_pad), lambda: (0, 0, 0)),
        ],
        out_specs=pl.BlockSpec((1, P_pad), lambda: (0, 0)),
        grid=(),
    )(r32, w, b, agg_parts)


def kernel(r_s, weight_W, bias, edge_index):
    N, F = r_s.shape
    P = weight_W.shape[1]
    E = edge_index.shape[1]
    k = P // 10

    P_pad = ((P + 127) // 128) * 128
    k_pad = min(((max(k, 1) + 127) // 128) * 128, P_pad)

    devs = [d for d in jax.devices() if d.platform == "tpu"]
    D = 2 if len(devs) >= 2 else 1

    nl = 1024
    while nl >= N and nl > 8:
        nl //= 2
    nh = N // nl

    E_TILE = 2048
    n_sub = 2
    n_steps = pl.cdiv(pl.cdiv(E, E_TILE), D)
    E_pad = D * n_steps * E_TILE

    r32 = r_s.astype(jnp.float32)
    w = jnp.zeros((F, P_pad), jnp.float32).at[:, :P].set(
        weight_W.astype(jnp.float32))
    b = jnp.zeros((1, P_pad), jnp.float32).at[:, :P].set(
        bias.astype(jnp.float32)[None, :])
    wk = w[:, :k_pad]
    row_flat = jnp.full((E_pad,), -1, jnp.int32).at[:E].set(
        edge_index[0].astype(jnp.int32))
    row = row_flat.reshape(D, n_steps * E_TILE)
    rowc = row_flat[:, None]
    col = jnp.zeros((E_pad,), jnp.int32).at[:E].set(
        edge_index[1].astype(jnp.int32))

    edge_call = functools.partial(_edge_call, N=N, F=F, k_pad=k_pad,
                                  E_TILE=E_TILE, n_steps=n_steps,
                                  n_sub=n_sub, nh=nh, nl=nl)
    fin_call = functools.partial(_finalize_call, N=N, F=F, P_pad=P_pad,
                                 k=k, k_pad=k_pad)

    if D == 1:
        agg = edge_call(r32, wk, row, rowc, col)
        out = fin_call(r32, w, b, agg[None])
        return out[0, :P]

    mesh = Mesh(np.array(devs[:D]), ("x",))

    def _sharded(r32, wk, w, b, row_sh, rowc_sh, col_sh):
        agg = edge_call(r32, wk, row_sh, rowc_sh, col_sh)
        agg_all = lax.all_gather(agg, "x")
        out = fin_call(r32, w, b, agg_all)
        return out

    out = _shard_map(
        _sharded, mesh=mesh,
        in_specs=(PSpec(None, None), PSpec(None, None),
                  PSpec(None, None), PSpec(None, None),
                  PSpec("x", None), PSpec("x", None), PSpec("x")),
        out_specs=PSpec("x", None),
        check_rep=False,
    )(r32, wk, w, b, row, rowc, col)
    return out[0, :P]


# speedup vs baseline: 7.4346x; 1.0474x over previous
import functools

import numpy as np

import jax
import jax.numpy as jnp
from jax import lax
from jax.experimental import pallas as pl
from jax.experimental.pallas import tpu as pltpu
from jax.sharding import Mesh, PartitionSpec as PSpec

try:
    from jax.experimental.shard_map import shard_map as _shard_map
except ImportError:
    _shard_map = jax.shard_map


def _edge_kernel(r_ref, wk_ref, row_ref, rowc_ref, col_ref, out_ref,
                 v1_ref, ev_ref, agg_ref, *, n_steps, n_sub, nh, nl, k_pad):
    e = pl.program_id(0)
    e_tile = row_ref.shape[1]
    sub = e_tile // n_sub

    @pl.when(e == 0)
    def _init():
        v128 = jnp.dot(r_ref[...], wk_ref[...],
                       preferred_element_type=jnp.float32)
        v1_ref[...] = v128[:, None, :]
        agg_ref[...] = jnp.zeros_like(agg_ref)

    for i in range(e_tile):
        ev_ref[i, :] = v1_ref[col_ref[i], 0, :]

    shift = jnp.int32(nl.bit_length() - 1)
    for s in range(n_sub):
        e_vals = ev_ref[s * sub:(s + 1) * sub, :].astype(jnp.bfloat16)

        rowv = rowc_ref[s * sub:(s + 1) * sub, :]
        rhi_b = jnp.broadcast_to(rowv, (sub, k_pad)) >> shift
        xs = [jnp.where(rhi_b == h, e_vals, jnp.bfloat16(0.0))
              for h in range(nh)]
        x = jnp.concatenate(xs, axis=1)

        rlo = row_ref[:, s * sub:(s + 1) * sub] & jnp.int32(nl - 1)
        rlo_iota = lax.broadcasted_iota(jnp.int32, (nl, sub), 0)
        rlo_oh = (rlo_iota == rlo).astype(jnp.bfloat16)
        agg_ref[...] += lax.dot_general(
            rlo_oh, x, dimension_numbers=(((1,), (0,)), ((), ())),
            preferred_element_type=jnp.float32)

    @pl.when(e == n_steps - 1)
    def _emit():
        for h in range(nh):
            out_ref[h * nl:(h + 1) * nl, :] = (
                agg_ref[:, h * k_pad:(h + 1) * k_pad])


def _finalize_kernel(r_ref, w_ref, b_ref, agg_ref, out_ref, *, k, k_pad):
    n_nodes = r_ref.shape[0]
    n_parts = agg_ref.shape[0]
    v = jnp.dot(r_ref[...], w_ref[...],
                preferred_element_type=jnp.float32)
    agg = agg_ref[0]
    for i in range(1, n_parts):
        agg = agg + agg_ref[i]
    lane = lax.broadcasted_iota(jnp.int32, (n_nodes, k_pad), 1)
    head = jnp.where(lane < k, agg, v[:, 0:k_pad])
    if k_pad < v.shape[1]:
        combined = jnp.concatenate([head, v[:, k_pad:]], axis=1)
    else:
        combined = head
    combined = combined + b_ref[...]
    i_s = jnp.maximum(combined, jnp.float32(0.0))
    out_ref[...] = jnp.mean(i_s, axis=0, keepdims=True)


def _edge_call(r32, wk, row, rowc, col, *, N, F, k_pad, E_TILE, n_steps,
               n_sub, nh, nl):
    body = functools.partial(_edge_kernel, n_steps=n_steps, n_sub=n_sub,
                             nh=nh, nl=nl, k_pad=k_pad)
    return pl.pallas_call(
        body,
        out_shape=jax.ShapeDtypeStruct((N, k_pad), jnp.float32),
        grid_spec=pltpu.PrefetchScalarGridSpec(
            num_scalar_prefetch=0,
            grid=(n_steps,),
            in_specs=[
                pl.BlockSpec((N, F), lambda e: (0, 0)),
                pl.BlockSpec((F, k_pad), lambda e: (0, 0)),
                pl.BlockSpec((1, E_TILE), lambda e: (0, e)),
                pl.BlockSpec((E_TILE, 1), lambda e: (e, 0)),
                pl.BlockSpec((E_TILE,), lambda e: (e,),
                             memory_space=pltpu.SMEM),
            ],
            out_specs=pl.BlockSpec((N, k_pad), lambda e: (0, 0)),
            scratch_shapes=[
                pltpu.VMEM((N, 1, k_pad), jnp.float32),
                pltpu.VMEM((E_TILE, k_pad), jnp.float32),
                pltpu.VMEM((nl, nh * k_pad), jnp.float32),
            ],
        ),
        compiler_params=pltpu.CompilerParams(
            dimension_semantics=("arbitrary",),
        ),
    )(r32, wk, row, rowc, col)


def _finalize_call(r32, w, b, agg_parts, *, N, F, P_pad, k, k_pad):
    n_parts = agg_parts.shape[0]
    return pl.pallas_call(
        functools.partial(_finalize_kernel, k=k, k_pad=k_pad),
        out_shape=jax.ShapeDtypeStruct((1, P_pad), jnp.float32),
        in_specs=[
            pl.BlockSpec((N, F), lambda: (0, 0)),
            pl.BlockSpec((F, P_pad), lambda: (0, 0)),
            pl.BlockSpec((1, P_pad), lambda: (0, 0)),
            pl.BlockSpec((n_parts, N, k_pad), lambda: (0, 0, 0)),
        ],
        out_specs=pl.BlockSpec((1, P_pad), lambda: (0, 0)),
        grid=(),
    )(r32, w, b, agg_parts)


def kernel(r_s, weight_W, bias, edge_index):
    N, F = r_s.shape
    P = weight_W.shape[1]
    E = edge_index.shape[1]
    k = P // 10

    P_pad = ((P + 127) // 128) * 128
    k_pad = min(((max(k, 1) + 127) // 128) * 128, P_pad)

    devs = [d for d in jax.devices() if d.platform == "tpu"]
    D = 2 if len(devs) >= 2 else 1

    nl = 1024
    while nl >= N and nl > 8:
        nl //= 2
    nh = N // nl

    E_TILE = 4096
    n_sub = 1
    n_steps = pl.cdiv(pl.cdiv(E, E_TILE), D)
    E_pad = D * n_steps * E_TILE

    r32 = r_s.astype(jnp.float32)
    w = jnp.zeros((F, P_pad), jnp.float32).at[:, :P].set(
        weight_W.astype(jnp.float32))
    b = jnp.zeros((1, P_pad), jnp.float32).at[:, :P].set(
        bias.astype(jnp.float32)[None, :])
    wk = w[:, :k_pad]
    row_flat = jnp.full((E_pad,), -1, jnp.int32).at[:E].set(
        edge_index[0].astype(jnp.int32))
    row = row_flat.reshape(D, n_steps * E_TILE)
    rowc = row_flat[:, None]
    col = jnp.zeros((E_pad,), jnp.int32).at[:E].set(
        edge_index[1].astype(jnp.int32))

    edge_call = functools.partial(_edge_call, N=N, F=F, k_pad=k_pad,
                                  E_TILE=E_TILE, n_steps=n_steps,
                                  n_sub=n_sub, nh=nh, nl=nl)
    fin_call = functools.partial(_finalize_call, N=N, F=F, P_pad=P_pad,
                                 k=k, k_pad=k_pad)

    if D == 1:
        agg = edge_call(r32, wk, row, rowc, col)
        out = fin_call(r32, w, b, agg[None])
        return out[0, :P]

    mesh = Mesh(np.array(devs[:D]), ("x",))

    def _sharded(r32, wk, w, b, row_sh, rowc_sh, col_sh):
        agg = edge_call(r32, wk, row_sh, rowc_sh, col_sh)
        agg_all = lax.all_gather(agg, "x")
        out = fin_call(r32, w, b, agg_all)
        return out

    out = _shard_map(
        _sharded, mesh=mesh,
        in_specs=(PSpec(None, None), PSpec(None, None),
                  PSpec(None, None), PSpec(None, None),
                  PSpec("x", None), PSpec("x", None), PSpec("x")),
        out_specs=PSpec("x", None),
        check_rep=False,
    )(r32, wk, w, b, row, rowc, col)
    return out[0, :P]


# speedup vs baseline: 7.4485x; 1.0019x over previous
import functools

import numpy as np

import jax
import jax.numpy as jnp
from jax import lax
from jax.experimental import pallas as pl
from jax.experimental.pallas import tpu as pltpu
from jax.sharding import Mesh, PartitionSpec as PSpec

try:
    from jax.experimental.shard_map import shard_map as _shard_map
except ImportError:
    _shard_map = jax.shard_map


def _edge_kernel(r_ref, wk_ref, row_ref, rowc_ref, col_ref, out_ref,
                 v1_ref, ev_ref, agg_ref, *, n_steps, n_sub, nh, nl, k_pad):
    e = pl.program_id(0)
    e_tile = row_ref.shape[1]
    sub = e_tile // n_sub

    @pl.when(e == 0)
    def _init():
        v128 = jnp.dot(r_ref[...], wk_ref[...],
                       preferred_element_type=jnp.float32)
        v1_ref[...] = v128[:, None, :]
        agg_ref[...] = jnp.zeros_like(agg_ref)

    for i in range(e_tile):
        ev_ref[i, :] = v1_ref[col_ref[i], 0, :]

    shift = jnp.int32(nl.bit_length() - 1)
    for s in range(n_sub):
        e_vals = ev_ref[s * sub:(s + 1) * sub, :].astype(jnp.bfloat16)

        rowv = rowc_ref[s * sub:(s + 1) * sub, :]
        rhi_b = jnp.broadcast_to(rowv, (sub, k_pad)) >> shift
        xs = [jnp.where(rhi_b == h, e_vals, jnp.bfloat16(0.0))
              for h in range(nh)]
        x = jnp.concatenate(xs, axis=1)

        rlo = row_ref[:, s * sub:(s + 1) * sub] & jnp.int32(nl - 1)
        rlo_iota = lax.broadcasted_iota(jnp.int32, (nl, sub), 0)
        rlo_oh = (rlo_iota == rlo).astype(jnp.bfloat16)
        agg_ref[...] += lax.dot_general(
            rlo_oh, x, dimension_numbers=(((1,), (0,)), ((), ())),
            preferred_element_type=jnp.float32)

    @pl.when(e == n_steps - 1)
    def _emit():
        for h in range(nh):
            out_ref[h * nl:(h + 1) * nl, :] = (
                agg_ref[:, h * k_pad:(h + 1) * k_pad])


def _finalize_kernel(r_ref, w_ref, b_ref, agg_ref, out_ref, *, k, k_pad):
    n_nodes = r_ref.shape[0]
    n_parts = agg_ref.shape[0]
    v = jnp.dot(r_ref[...], w_ref[...],
                preferred_element_type=jnp.float32)
    agg = agg_ref[0]
    for i in range(1, n_parts):
        agg = agg + agg_ref[i]
    lane = lax.broadcasted_iota(jnp.int32, (n_nodes, k_pad), 1)
    head = jnp.where(lane < k, agg, v[:, 0:k_pad])
    if k_pad < v.shape[1]:
        combined = jnp.concatenate([head, v[:, k_pad:]], axis=1)
    else:
        combined = head
    combined = combined + b_ref[...]
    i_s = jnp.maximum(combined, jnp.float32(0.0))
    out_ref[...] = jnp.mean(i_s, axis=0, keepdims=True)


def _edge_call(r32, wk, row, rowc, col, *, N, F, k_pad, E_TILE, n_steps,
               n_sub, nh, nl):
    body = functools.partial(_edge_kernel, n_steps=n_steps, n_sub=n_sub,
                             nh=nh, nl=nl, k_pad=k_pad)
    return pl.pallas_call(
        body,
        out_shape=jax.ShapeDtypeStruct((N, k_pad), jnp.float32),
        grid_spec=pltpu.PrefetchScalarGridSpec(
            num_scalar_prefetch=0,
            grid=(n_steps,),
            in_specs=[
                pl.BlockSpec((N, F), lambda e: (0, 0)),
                pl.BlockSpec((F, k_pad), lambda e: (0, 0)),
                pl.BlockSpec((1, E_TILE), lambda e: (0, e)),
                pl.BlockSpec((E_TILE, 1), lambda e: (e, 0)),
                pl.BlockSpec((E_TILE,), lambda e: (e,),
                             memory_space=pltpu.SMEM),
            ],
            out_specs=pl.BlockSpec((N, k_pad), lambda e: (0, 0)),
            scratch_shapes=[
                pltpu.VMEM((N, 1, k_pad), jnp.float32),
                pltpu.VMEM((E_TILE, k_pad), jnp.float32),
                pltpu.VMEM((nl, nh * k_pad), jnp.float32),
            ],
        ),
        compiler_params=pltpu.CompilerParams(
            dimension_semantics=("arbitrary",),
        ),
    )(r32, wk, row, rowc, col)


def _finalize_call(r32, w, b, agg_parts, *, N, F, P_pad, k, k_pad):
    n_parts = agg_parts.shape[0]
    return pl.pallas_call(
        functools.partial(_finalize_kernel, k=k, k_pad=k_pad),
        out_shape=jax.ShapeDtypeStruct((1, P_pad), jnp.float32),
        in_specs=[
            pl.BlockSpec((N, F), lambda: (0, 0)),
            pl.BlockSpec((F, P_pad), lambda: (0, 0)),
            pl.BlockSpec((1, P_pad), lambda: (0, 0)),
            pl.BlockSpec((n_parts, N, k_pad), lambda: (0, 0, 0)),
        ],
        out_specs=pl.BlockSpec((1, P_pad), lambda: (0, 0)),
        grid=(),
    )(r32, w, b, agg_parts)


def kernel(r_s, weight_W, bias, edge_index):
    N, F = r_s.shape
    P = weight_W.shape[1]
    E = edge_index.shape[1]
    k = P // 10

    P_pad = ((P + 127) // 128) * 128
    k_pad = min(((max(k, 1) + 127) // 128) * 128, P_pad)

    devs = [d for d in jax.devices() if d.platform == "tpu"]
    D = 2 if len(devs) >= 2 else 1

    nl = 1024
    while nl >= N and nl > 8:
        nl //= 2
    nh = N // nl

    E_TILE = 8192
    n_sub = 1
    n_steps = pl.cdiv(pl.cdiv(E, E_TILE), D)
    E_pad = D * n_steps * E_TILE

    r32 = r_s.astype(jnp.float32)
    w = jnp.zeros((F, P_pad), jnp.float32).at[:, :P].set(
        weight_W.astype(jnp.float32))
    b = jnp.zeros((1, P_pad), jnp.float32).at[:, :P].set(
        bias.astype(jnp.float32)[None, :])
    wk = w[:, :k_pad]
    row_flat = jnp.full((E_pad,), -1, jnp.int32).at[:E].set(
        edge_index[0].astype(jnp.int32))
    row = row_flat.reshape(D, n_steps * E_TILE)
    rowc = row_flat[:, None]
    col = jnp.zeros((E_pad,), jnp.int32).at[:E].set(
        edge_index[1].astype(jnp.int32))

    edge_call = functools.partial(_edge_call, N=N, F=F, k_pad=k_pad,
                                  E_TILE=E_TILE, n_steps=n_steps,
                                  n_sub=n_sub, nh=nh, nl=nl)
    fin_call = functools.partial(_finalize_call, N=N, F=F, P_pad=P_pad,
                                 k=k, k_pad=k_pad)

    if D == 1:
        agg = edge_call(r32, wk, row, rowc, col)
        out = fin_call(r32, w, b, agg[None])
        return out[0, :P]

    mesh = Mesh(np.array(devs[:D]), ("x",))

    def _sharded(r32, wk, w, b, row_sh, rowc_sh, col_sh):
        agg = edge_call(r32, wk, row_sh, rowc_sh, col_sh)
        agg_all = lax.all_gather(agg, "x")
        out = fin_call(r32, w, b, agg_all)
        return out

    out = _shard_map(
        _sharded, mesh=mesh,
        in_specs=(PSpec(None, None), PSpec(None, None),
                  PSpec(None, None), PSpec(None, None),
                  PSpec("x", None), PSpec("x", None), PSpec("x")),
        out_specs=PSpec("x", None),
        check_rep=False,
    )(r32, wk, w, b, row, rowc, col)
    return out[0, :P]


# speedup vs baseline: 7.6508x; 1.0272x over previous
import functools

import numpy as np

import jax
import jax.numpy as jnp
from jax import lax
from jax.experimental import pallas as pl
from jax.experimental.pallas import tpu as pltpu
from jax.sharding import Mesh, PartitionSpec as PSpec

try:
    from jax.experimental.shard_map import shard_map as _shard_map
except ImportError:
    _shard_map = jax.shard_map


def _edge_kernel(r_ref, wk_ref, row_ref, rowc_ref, col_ref, out_ref,
                 v1_ref, ev_ref, agg_ref, *, n_steps, n_sub, nh, nl, k_pad):
    e = pl.program_id(0)
    e_tile = row_ref.shape[1]
    sub = e_tile // n_sub

    @pl.when(e == 0)
    def _init():
        v128 = jnp.dot(r_ref[...], wk_ref[...],
                       preferred_element_type=jnp.float32)
        v1_ref[...] = v128[:, None, :]
        agg_ref[...] = jnp.zeros_like(agg_ref)

    for i in range(e_tile):
        ev_ref[i, :] = v1_ref[col_ref[i], 0, :]

    shift = jnp.int32(nl.bit_length() - 1)
    for s in range(n_sub):
        e_vals = ev_ref[s * sub:(s + 1) * sub, :].astype(jnp.bfloat16)

        rowv = rowc_ref[s * sub:(s + 1) * sub, :]
        rhi_b = jnp.broadcast_to(rowv, (sub, k_pad)) >> shift
        xs = [jnp.where(rhi_b == h, e_vals, jnp.bfloat16(0.0))
              for h in range(nh)]
        x = jnp.concatenate(xs, axis=1)

        rlo = row_ref[:, s * sub:(s + 1) * sub] & jnp.int32(nl - 1)
        rlo_iota = lax.broadcasted_iota(jnp.int32, (nl, sub), 0)
        rlo_oh = (rlo_iota == rlo).astype(jnp.bfloat16)
        agg_ref[...] += lax.dot_general(
            rlo_oh, x, dimension_numbers=(((1,), (0,)), ((), ())),
            preferred_element_type=jnp.float32)

    @pl.when(e == n_steps - 1)
    def _emit():
        for h in range(nh):
            out_ref[h * nl:(h + 1) * nl, :] = (
                agg_ref[:, h * k_pad:(h + 1) * k_pad])


def _finalize_kernel(r_ref, w_ref, b_ref, agg_ref, out_ref, *, k, k_pad):
    n_nodes = r_ref.shape[0]
    n_parts = agg_ref.shape[0]
    v = jnp.dot(r_ref[...], w_ref[...],
                preferred_element_type=jnp.float32)
    agg = agg_ref[0]
    for i in range(1, n_parts):
        agg = agg + agg_ref[i]
    lane = lax.broadcasted_iota(jnp.int32, (n_nodes, k_pad), 1)
    head = jnp.where(lane < k, agg, v[:, 0:k_pad])
    if k_pad < v.shape[1]:
        combined = jnp.concatenate([head, v[:, k_pad:]], axis=1)
    else:
        combined = head
    combined = combined + b_ref[...]
    i_s = jnp.maximum(combined, jnp.float32(0.0))
    out_ref[...] = jnp.mean(i_s, axis=0, keepdims=True)


def _edge_call(r32, wk, row, rowc, col, *, N, F, k_pad, E_TILE, n_steps,
               n_sub, nh, nl):
    body = functools.partial(_edge_kernel, n_steps=n_steps, n_sub=n_sub,
                             nh=nh, nl=nl, k_pad=k_pad)
    return pl.pallas_call(
        body,
        out_shape=jax.ShapeDtypeStruct((N, k_pad), jnp.float32),
        grid_spec=pltpu.PrefetchScalarGridSpec(
            num_scalar_prefetch=0,
            grid=(n_steps,),
            in_specs=[
                pl.BlockSpec((N, F), lambda e: (0, 0)),
                pl.BlockSpec((F, k_pad), lambda e: (0, 0)),
                pl.BlockSpec((1, E_TILE), lambda e: (0, e)),
                pl.BlockSpec((E_TILE, 1), lambda e: (e, 0)),
                pl.BlockSpec((E_TILE,), lambda e: (e,),
                             memory_space=pltpu.SMEM),
            ],
            out_specs=pl.BlockSpec((N, k_pad), lambda e: (0, 0)),
            scratch_shapes=[
                pltpu.VMEM((N, 1, k_pad), jnp.float32),
                pltpu.VMEM((E_TILE, k_pad), jnp.float32),
                pltpu.VMEM((nl, nh * k_pad), jnp.float32),
            ],
        ),
        compiler_params=pltpu.CompilerParams(
            dimension_semantics=("arbitrary",),
        ),
    )(r32, wk, row, rowc, col)


def _finalize_call(r32, w, b, agg_parts, *, N, F, P_pad, k, k_pad):
    n_parts = agg_parts.shape[0]
    return pl.pallas_call(
        functools.partial(_finalize_kernel, k=k, k_pad=k_pad),
        out_shape=jax.ShapeDtypeStruct((1, P_pad), jnp.float32),
        in_specs=[
            pl.BlockSpec((N, F), lambda: (0, 0)),
            pl.BlockSpec((F, P_pad), lambda: (0, 0)),
            pl.BlockSpec((1, P_pad), lambda: (0, 0)),
            pl.BlockSpec((n_parts, N, k_pad), lambda: (0, 0, 0)),
        ],
        out_specs=pl.BlockSpec((1, P_pad), lambda: (0, 0)),
        grid=(),
    )(r32, w, b, agg_parts)


def kernel(r_s, weight_W, bias, edge_index):
    N, F = r_s.shape
    P = weight_W.shape[1]
    E = edge_index.shape[1]
    k = P // 10

    P_pad = ((P + 127) // 128) * 128
    k_pad = min(((max(k, 1) + 127) // 128) * 128, P_pad)

    devs = [d for d in jax.devices() if d.platform == "tpu"]
    D = 2 if len(devs) >= 2 else 1

    nl = 1024
    while nl >= N and nl > 8:
        nl //= 2
    nh = N // nl

    E_TILE = 4096
    n_sub = 1
    n_steps = pl.cdiv(pl.cdiv(E, E_TILE), D)
    E_pad = D * n_steps * E_TILE

    r32 = r_s.astype(jnp.float32)
    w = jnp.zeros((F, P_pad), jnp.float32).at[:, :P].set(
        weight_W.astype(jnp.float32))
    b = jnp.zeros((1, P_pad), jnp.float32).at[:, :P].set(
        bias.astype(jnp.float32)[None, :])
    wk = w[:, :k_pad]
    row_flat = jnp.full((E_pad,), -1, jnp.int32).at[:E].set(
        edge_index[0].astype(jnp.int32))
    row = row_flat.reshape(D, n_steps * E_TILE)
    rowc = row_flat[:, None]
    col = jnp.zeros((E_pad,), jnp.int32).at[:E].set(
        edge_index[1].astype(jnp.int32))

    edge_call = functools.partial(_edge_call, N=N, F=F, k_pad=k_pad,
                                  E_TILE=E_TILE, n_steps=n_steps,
                                  n_sub=n_sub, nh=nh, nl=nl)
    fin_call = functools.partial(_finalize_call, N=N, F=F, P_pad=P_pad,
                                 k=k, k_pad=k_pad)

    if D == 1:
        agg = edge_call(r32, wk, row, rowc, col)
        out = fin_call(r32, w, b, agg[None])
        return out[0, :P]

    mesh = Mesh(np.array(devs[:D]), ("x",))

    def _sharded(r32, wk, w, b, row_sh, rowc_sh, col_sh):
        agg = edge_call(r32, wk, row_sh, rowc_sh, col_sh)
        agg_all = lax.all_gather(agg, "x")
        out = fin_call(r32, w, b, agg_all)
        return out

    out = _shard_map(
        _sharded, mesh=mesh,
        in_specs=(PSpec(None, None), PSpec(None, None),
                  PSpec(None, None), PSpec(None, None),
                  PSpec("x", None), PSpec("x", None), PSpec("x")),
        out_specs=PSpec("x", None),
        check_rep=False,
    )(r32, wk, w, b, row, rowc, col)
    return out[0, :P]
